# baseline (speedup 1.0000x reference)
import functools

import jax
import jax.numpy as jnp
from jax import lax
from jax.experimental import pallas as pl
from jax.experimental.pallas import tpu as pltpu
from jax.experimental.pallas import tpu_sc as plsc

N = 10000
E = 320000
D_IN = 128
D_EDGE = 16
H = 128
G = 256

NC = 2
NS = 16
NW = NC * NS
CH = 64
EPW = 10240
E_PAD = NW * EPW
NCH = EPW // CH
N_PAD = 10112
RPS = N_PAD // NS


def _matmul_bias_body(a_ref, w_ref, b_ref, o_ref):
    o_ref[...] = (
        jnp.dot(a_ref[...], w_ref[...], preferred_element_type=jnp.float32)
        + b_ref[...]
    )


def _node_proj(x, w, b):
    return pl.pallas_call(
        _matmul_bias_body,
        out_shape=jax.ShapeDtypeStruct((N, H), jnp.float32),
    )(x, w, b.reshape(1, H))


def _edge_proj(ea, w, b):
    be = 8192
    return pl.pallas_call(
        _matmul_bias_body,
        grid=(E_PAD // be,),
        in_specs=[
            pl.BlockSpec((be, D_EDGE), lambda i: (i, 0)),
            pl.BlockSpec((D_EDGE, H), lambda i: (0, 0)),
            pl.BlockSpec((1, H), lambda i: (0, 0)),
        ],
        out_specs=pl.BlockSpec((be, H), lambda i: (i, 0)),
        out_shape=jax.ShapeDtypeStruct((E_PAD, H), jnp.float32),
    )(ea, w, b.reshape(1, H))


def _layer_body(h_ref, p0_ref, p1_ref, eps_ref, gamma_ref, beta_ref,
                w0_ref, b0_ref, w1_ref, b1_ref, w2_ref, b2_ref, w3_ref, b3_ref,
                o_ref):
    h = h_ref[...]
    t = (1.0 + eps_ref[...]) * h + (p0_ref[...] + p1_ref[...])
    for i, (w_r, b_r) in enumerate(
        ((w0_ref, b0_ref), (w1_ref, b1_ref), (w2_ref, b2_ref), (w3_ref, b3_ref))
    ):
        t = jnp.dot(t, w_r[...], preferred_element_type=jnp.float32) + b_r[...]
        if i < 3:
            t = jnp.maximum(t, 0.0)
    mean = jnp.mean(t, axis=0, keepdims=True)
    c = t - mean
    var = jnp.mean(c * c, axis=0, keepdims=True)
    t = c / jnp.sqrt(var + 1e-5) * gamma_ref[...] + beta_ref[...]
    o_ref[...] = jnp.maximum(t, 0.0) + h


def _dense_layer(h, p0, p1, lp):
    mlp = lp['mlp']
    args = [h, p0, p1, lp['eps'].reshape(1, 1), lp['gamma'].reshape(1, H),
            lp['beta'].reshape(1, H)]
    for w, b in mlp:
        args.append(w)
        args.append(b.reshape(1, H))
    return pl.pallas_call(
        _layer_body,
        out_shape=jax.ShapeDtypeStruct((N, H), jnp.float32),
    )(*args)


def _pool_body(h_ref, b_ref, o_ref):
    gid = lax.broadcasted_iota(jnp.int32, (G, N), 0)
    onehot = (gid == b_ref[...]).astype(jnp.float32)
    sums = jnp.dot(onehot, h_ref[...], preferred_element_type=jnp.float32)
    counts = jnp.sum(onehot, axis=1, keepdims=True)
    o_ref[...] = sums / jnp.maximum(counts, 1.0)


def _pool(h, batch_row):
    return pl.pallas_call(
        _pool_body,
        out_shape=jax.ShapeDtypeStruct((G, H), jnp.float32),
    )(h, batch_row)


def _sc_aggr(h, e, src, dst):
    mesh = plsc.VectorSubcoreMesh(core_axis_name="c", subcore_axis_name="s")

    @functools.partial(
        pl.kernel,
        out_type=(
            jax.ShapeDtypeStruct((N_PAD, H), jnp.float32),
            jax.ShapeDtypeStruct((N_PAD, H), jnp.float32),
        ),
        mesh=mesh,
        compiler_params=pltpu.CompilerParams(use_tc_tiling_on_sc=False),
        scratch_types=[
            pltpu.VMEM((NCH, CH), jnp.int32),
            pltpu.VMEM((NCH, CH), jnp.int32),
            pltpu.VMEM((CH, H), jnp.float32),
            pltpu.VMEM((CH, H), jnp.float32),
            pltpu.VMEM((CH, H), jnp.float32),
            pltpu.VMEM_SHARED((N_PAD, H), jnp.float32),
            pltpu.SemaphoreType.DMA,
            pltpu.SemaphoreType.DMA,
            pltpu.SemaphoreType.DMA,
        ],
    )
    def k(h_hbm, e_hbm, src_hbm, dst_hbm, out0, out1,
          src_all, dst_all, e0, g0, g1, acc_sh, sem_e, sem_g0, sem_g1):
        cid = lax.axis_index("c")
        sid = lax.axis_index("s")
        w = cid * NS + sid

        pltpu.sync_copy(src_hbm.at[w], src_all)
        pltpu.sync_copy(dst_hbm.at[w], dst_all)

        def zrow(i, _):
            for j in range(H // 16):
                e0[i, pl.ds(j * 16, 16)] = jnp.zeros((16,), jnp.float32)
            return 0
        lax.fori_loop(0, CH, zrow, 0)

        def zblk(i, _):
            pltpu.sync_copy(e0, acc_sh.at[pl.ds(sid * RPS + i * CH, CH)])
            return 0
        lax.fori_loop(0, RPS // CH, zblk, 0)
        rem = RPS % CH
        if rem:
            pltpu.sync_copy(
                e0.at[pl.ds(0, rem)],
                acc_sh.at[pl.ds(sid * RPS + (RPS // CH) * CH, rem)])
        plsc.subcore_barrier()

        base = w * EPW

        def issue_g(t, gv, sg):
            pltpu.async_copy(h_hbm.at[src_all.at[t]], gv, sg)

        def wait_g(t, gv, sg):
            pltpu.make_async_copy(h_hbm.at[src_all.at[t]], gv, sg).wait()

        def issue_e(t):
            pltpu.async_copy(e_hbm.at[pl.ds(base + t * CH, CH)], e0, sem_e)

        def wait_e():
            pltpu.make_async_copy(e_hbm.at[pl.ds(0, CH)], e0, sem_e).wait()

        def compute_scatter(t, gv):
            def row(i, _):
                for j in range(H // 16):
                    s = pl.ds(j * 16, 16)
                    gv[i, s] = jnp.maximum(gv[i, s] + e0[i, s], 0.0)
                return 0
            lax.fori_loop(0, CH, row, 0)
            pltpu.sync_copy(gv, acc_sh.at[dst_all.at[t]], add=True)

        issue_g(0, g0, sem_g0)
        issue_e(0)

        def pipe(i, _):
            t = 2 * i
            issue_g(t + 1, g1, sem_g1)
            wait_e()
            wait_g(t, g0, sem_g0)
            compute_scatter(t, g0)
            issue_e(t + 1)

            @pl.when(i < NCH // 2 - 1)
            def _():
                issue_g(t + 2, g0, sem_g0)
            wait_e()
            wait_g(t + 1, g1, sem_g1)
            compute_scatter(t + 1, g1)

            @pl.when(i < NCH // 2 - 1)
            def _():
                issue_e(t + 2)
            return 0
        lax.fori_loop(0, NCH // 2, pipe, 0)
        plsc.subcore_barrier()

        rows = pl.ds(sid * RPS, RPS)

        @pl.when(cid == 0)
        def _():
            pltpu.sync_copy(acc_sh.at[rows], out0.at[rows])

        @pl.when(cid == 1)
        def _():
            pltpu.sync_copy(acc_sh.at[rows], out1.at[rows])

    return k(h, e, src, dst)


def kernel(x, edge_index, batch, edge_attr, params):
    src = edge_index[0]
    dst = edge_index[1]
    pad = E_PAD - E
    src_p = jnp.concatenate([src, jnp.zeros((pad,), jnp.int32)])
    trash = N + jnp.arange(pad, dtype=jnp.int32) % (N_PAD - N)
    dst_p = jnp.concatenate([dst, trash])
    ea_p = jnp.concatenate([edge_attr, jnp.zeros((pad, D_EDGE), jnp.float32)])

    p = params
    h = _node_proj(x, p['node_W'], p['node_b'])
    e = _edge_proj(ea_p, p['edge_W'], p['edge_b'])
    src_3d = src_p.reshape(NW, NCH, CH)
    dst_3d = dst_p.reshape(NW, NCH, CH)
    for lp in p['layers']:
        a0, a1 = _sc_aggr(h, e, src_3d, dst_3d)
        h = _dense_layer(h, a0[:N], a1[:N], lp)
    return _pool(h, batch.reshape(1, N))


# speedup vs baseline: 1.4784x; 1.4784x over previous
import functools

import jax
import jax.numpy as jnp
from jax import lax
from jax.experimental import pallas as pl
from jax.experimental.pallas import tpu as pltpu
from jax.experimental.pallas import tpu_sc as plsc

N = 10000
E = 320000
D_IN = 128
D_EDGE = 16
H = 128
HH = H // 2
G = 256

NC = 2
NS = 16
CH = 128
EPW = 20352
E_PAD = NS * EPW
NCH = EPW // CH
N_PAD = 10112
RPS = N_PAD // NS


def _proj_split_body(a_ref, w_ref, b_ref, o_ref, o2_ref):
    t = (jnp.dot(a_ref[...], w_ref[...], preferred_element_type=jnp.float32)
         + b_ref[...])
    o_ref[...] = t
    o2_ref[0, ...] = t[:, :HH]
    o2_ref[1, ...] = t[:, HH:]


def _split_only_body(a_ref, w_ref, b_ref, o2_ref):
    t = (jnp.dot(a_ref[...], w_ref[...], preferred_element_type=jnp.float32)
         + b_ref[...])
    o2_ref[0, ...] = t[:, :HH]
    o2_ref[1, ...] = t[:, HH:]


def _node_proj(x, w, b):
    return pl.pallas_call(
        _proj_split_body,
        out_shape=(
            jax.ShapeDtypeStruct((N, H), jnp.float32),
            jax.ShapeDtypeStruct((NC, N, HH), jnp.float32),
        ),
    )(x, w, b.reshape(1, H))


def _edge_proj(ea, w, b):
    be = 6144
    return pl.pallas_call(
        _split_only_body,
        grid=(E_PAD // be,),
        in_specs=[
            pl.BlockSpec((be, D_EDGE), lambda i: (i, 0)),
            pl.BlockSpec((D_EDGE, H), lambda i: (0, 0)),
            pl.BlockSpec((1, H), lambda i: (0, 0)),
        ],
        out_specs=pl.BlockSpec((NC, be, HH), lambda i: (0, i, 0)),
        out_shape=jax.ShapeDtypeStruct((NC, E_PAD, HH), jnp.float32),
    )(ea, w, b.reshape(1, H))


def _layer_body(h_ref, p0_ref, p1_ref, eps_ref, gamma_ref, beta_ref,
                w0_ref, b0_ref, w1_ref, b1_ref, w2_ref, b2_ref, w3_ref, b3_ref,
                o_ref, o2_ref):
    h = h_ref[...]
    aggr = jnp.concatenate([p0_ref[...], p1_ref[...]], axis=1)
    t = (1.0 + eps_ref[...]) * h + aggr
    for i, (w_r, b_r) in enumerate(
        ((w0_ref, b0_ref), (w1_ref, b1_ref), (w2_ref, b2_ref), (w3_ref, b3_ref))
    ):
        t = jnp.dot(t, w_r[...], preferred_element_type=jnp.float32) + b_r[...]
        if i < 3:
            t = jnp.maximum(t, 0.0)
    mean = jnp.mean(t, axis=0, keepdims=True)
    c = t - mean
    var = jnp.mean(c * c, axis=0, keepdims=True)
    t = c / jnp.sqrt(var + 1e-5) * gamma_ref[...] + beta_ref[...]
    t = jnp.maximum(t, 0.0) + h
    o_ref[...] = t
    o2_ref[0, ...] = t[:, :HH]
    o2_ref[1, ...] = t[:, HH:]


def _dense_layer(h, p0, p1, lp):
    mlp = lp['mlp']
    args = [h, p0, p1, lp['eps'].reshape(1, 1), lp['gamma'].reshape(1, H),
            lp['beta'].reshape(1, H)]
    for w, b in mlp:
        args.append(w)
        args.append(b.reshape(1, H))
    return pl.pallas_call(
        _layer_body,
        out_shape=(
            jax.ShapeDtypeStruct((N, H), jnp.float32),
            jax.ShapeDtypeStruct((NC, N, HH), jnp.float32),
        ),
    )(*args)


def _pool_body(h_ref, b_ref, o_ref):
    gid = lax.broadcasted_iota(jnp.int32, (G, N), 0)
    onehot = (gid == b_ref[...]).astype(jnp.float32)
    sums = jnp.dot(onehot, h_ref[...], preferred_element_type=jnp.float32)
    counts = jnp.sum(onehot, axis=1, keepdims=True)
    o_ref[...] = sums / jnp.maximum(counts, 1.0)


def _pool(h, batch_row):
    return pl.pallas_call(
        _pool_body,
        out_shape=jax.ShapeDtypeStruct((G, H), jnp.float32),
    )(h, batch_row)


def _sc_aggr(h2, e2, src, dst):
    mesh = plsc.VectorSubcoreMesh(core_axis_name="c", subcore_axis_name="s")

    @functools.partial(
        pl.kernel,
        out_type=(
            jax.ShapeDtypeStruct((N_PAD, HH), jnp.float32),
            jax.ShapeDtypeStruct((N_PAD, HH), jnp.float32),
        ),
        mesh=mesh,
        compiler_params=pltpu.CompilerParams(use_tc_tiling_on_sc=False),
        scratch_types=[
            pltpu.VMEM((NCH, CH), jnp.int32),
            pltpu.VMEM((NCH, CH), jnp.int32),
            pltpu.VMEM((CH, HH), jnp.float32),
            pltpu.VMEM((CH, HH), jnp.float32),
            pltpu.VMEM((CH, HH), jnp.float32),
            pltpu.VMEM((CH, HH), jnp.float32),
            pltpu.VMEM((CH, HH), jnp.float32),
            pltpu.VMEM((CH, HH), jnp.float32),
            pltpu.VMEM_SHARED((N_PAD, HH), jnp.float32),
            pltpu.SemaphoreType.DMA,
            pltpu.SemaphoreType.DMA,
            pltpu.SemaphoreType.DMA,
            pltpu.SemaphoreType.DMA,
            pltpu.SemaphoreType.DMA,
            pltpu.SemaphoreType.DMA,
            pltpu.SemaphoreType.DMA,
            pltpu.SemaphoreType.DMA,
            pltpu.SemaphoreType.DMA,
        ],
    )
    def k(h2_hbm, e2_hbm, src_hbm, dst_hbm, out0, out1,
          src_all, dst_all, e0, e1, e2b, g0, g1, g2, acc_sh,
          sem_e0, sem_e1, sem_e2, sem_g0, sem_g1, sem_g2,
          sem_s0, sem_s1, sem_s2):
        cid = lax.axis_index("c")
        sid = lax.axis_index("s")
        h_c = h2_hbm.at[cid]
        e_c = e2_hbm.at[cid]

        pltpu.sync_copy(src_hbm.at[sid], src_all)
        pltpu.sync_copy(dst_hbm.at[sid], dst_all)

        def zrow(i, _):
            for j in range(HH // 16):
                e0[i, pl.ds(j * 16, 16)] = jnp.zeros((16,), jnp.float32)
            return 0
        lax.fori_loop(0, CH, zrow, 0)

        def zblk(i, _):
            pltpu.sync_copy(e0, acc_sh.at[pl.ds(sid * RPS + i * CH, CH)])
            return 0
        lax.fori_loop(0, RPS // CH, zblk, 0)
        rem = RPS % CH
        if rem:
            pltpu.sync_copy(
                e0.at[pl.ds(0, rem)],
                acc_sh.at[pl.ds(sid * RPS + (RPS // CH) * CH, rem)])
        plsc.subcore_barrier()

        base = sid * EPW
        bufs = ((e0, g0, sem_e0, sem_g0, sem_s0),
                (e1, g1, sem_e1, sem_g1, sem_s1),
                (e2b, g2, sem_e2, sem_g2, sem_s2))

        def issue(t, b):
            ev, gv, se, sg, _ = b
            pltpu.async_copy(e_c.at[pl.ds(base + t * CH, CH)], ev, se)
            pltpu.async_copy(h_c.at[src_all.at[t]], gv, sg)

        def wait_buf(t, b):
            ev, gv, se, sg, _ = b
            pltpu.make_async_copy(e_c.at[pl.ds(0, CH)], ev, se).wait()
            pltpu.make_async_copy(h_c.at[src_all.at[t]], gv, sg).wait()

        def wait_scatter(b):
            _, gv, _, _, ss = b
            pltpu.make_async_copy(gv, acc_sh.at[dst_all.at[0]], ss).wait()

        def compute_scatter(t, b):
            ev, gv, _, _, ss = b

            def row(i, _):
                for j in range(HH // 16):
                    s = pl.ds(j * 16, 16)
                    gv[i, s] = jnp.maximum(gv[i, s] + ev[i, s], 0.0)
                return 0
            lax.fori_loop(0, CH, row, 0)
            pltpu.async_copy(gv, acc_sh.at[dst_all.at[t]], ss, add=True)

        issue(0, bufs[0])
        issue(1, bufs[1])

        def pipe(i, _):
            for kk in range(3):
                t = 3 * i + kk
                b = bufs[kk]
                bn = bufs[(kk + 2) % 3]
                wait_buf(t, b)
                compute_scatter(t, b)

                @pl.when(t + 2 < NCH)
                def _():
                    @pl.when(t >= 1)
                    def _():
                        wait_scatter(bn)
                    issue(t + 2, bn)
            return 0
        lax.fori_loop(0, NCH // 3, pipe, 0)
        wait_scatter(bufs[0])
        wait_scatter(bufs[1])
        wait_scatter(bufs[2])
        plsc.subcore_barrier()

        rows = pl.ds(sid * RPS, RPS)

        @pl.when(cid == 0)
        def _():
            pltpu.sync_copy(acc_sh.at[rows], out0.at[rows])

        @pl.when(cid == 1)
        def _():
            pltpu.sync_copy(acc_sh.at[rows], out1.at[rows])

    return k(h2, e2, src, dst)


def kernel(x, edge_index, batch, edge_attr, params):
    src = edge_index[0]
    dst = edge_index[1]
    pad = E_PAD - E
    src_p = jnp.concatenate([src, jnp.zeros((pad,), jnp.int32)])
    trash = N + jnp.arange(pad, dtype=jnp.int32) % (N_PAD - N)
    dst_p = jnp.concatenate([dst, trash])
    ea_p = jnp.concatenate([edge_attr, jnp.zeros((pad, D_EDGE), jnp.float32)])

    p = params
    h, h2 = _node_proj(x, p['node_W'], p['node_b'])
    e2 = _edge_proj(ea_p, p['edge_W'], p['edge_b'])
    src_3d = src_p.reshape(NS, NCH, CH)
    dst_3d = dst_p.reshape(NS, NCH, CH)
    for lp in p['layers']:
        a0, a1 = _sc_aggr(h2, e2, src_3d, dst_3d)
        h, h2 = _dense_layer(h, a0[:N], a1[:N], lp)
    return _pool(h, batch.reshape(1, N))
